# baseline (speedup 1.0000x reference)
import functools

import jax
import jax.numpy as jnp
from jax import lax
from jax.experimental import pallas as pl
from jax.experimental.pallas import tpu as pltpu
from jax.experimental.pallas import tpu_sc as plsc

LANES = 16
CSTRIP = 16


def _iota16():
    return lax.iota(jnp.int32, LANES)


def _make_phase1(rows, cols, nw, num_cores, chunk_rows):
    rpw = rows // nw
    nchunks = rpw // chunk_rows
    ngroups = cols // LANES

    mesh = plsc.VectorSubcoreMesh(core_axis_name="c", subcore_axis_name="s")

    @functools.partial(
        pl.kernel,
        out_type=jax.ShapeDtypeStruct((nw * cols,), jnp.float32),
        mesh=mesh,
        scratch_types=[
            pltpu.VMEM((chunk_rows, cols), jnp.float32),
            pltpu.VMEM((chunk_rows, cols), jnp.float32),
            pltpu.VMEM((cols,), jnp.float32),
            pltpu.SemaphoreType.DMA,
            pltpu.SemaphoreType.DMA,
        ],
        compiler_params=pltpu.CompilerParams(needs_layout_passes=False),
    )
    def phase1(x_hbm, sums_hbm, xbuf0, xbuf1, carry, s0, s1):
        cid = lax.axis_index("c")
        sid = lax.axis_index("s")
        wid = sid * num_cores + cid
        row_base = wid * rpw

        def in_copy(c, buf, sem):
            return pltpu.make_async_copy(
                x_hbm.at[pl.ds(row_base + c * chunk_rows, chunk_rows), :],
                buf,
                sem,
            )

        def compute(xbuf, first):
            for cg in range(ngroups // CSTRIP):
                col0 = cg * CSTRIP * LANES
                cidx = [_iota16() + (col0 + g * LANES) for g in range(CSTRIP)]
                cs = [
                    jnp.where(
                        first,
                        jnp.zeros((LANES,), jnp.float32),
                        carry[pl.ds(col0 + g * LANES, LANES)],
                    )
                    for g in range(CSTRIP)
                ]

                def row_body(r, cs, _cidx=cidx):
                    ridx = jnp.full((LANES,), r, jnp.int32)
                    return tuple(
                        cs[g] + plsc.load_gather(xbuf, [ridx, _cidx[g]])
                        for g in range(CSTRIP)
                    )

                cs = lax.fori_loop(0, chunk_rows, row_body, tuple(cs), unroll=4)
                for g in range(CSTRIP):
                    carry[pl.ds(col0 + g * LANES, LANES)] = cs[g]

        in_copy(0, xbuf0, s0).start()

        def pair_body(i, _):
            c0 = 2 * i
            in_copy(c0 + 1, xbuf1, s1).start()
            in_copy(c0, xbuf0, s0).wait()
            compute(xbuf0, c0 == 0)

            @pl.when(c0 + 2 < nchunks)
            def _():
                in_copy(c0 + 2, xbuf0, s0).start()

            in_copy(c0 + 1, xbuf1, s1).wait()
            compute(xbuf1, False)
            return 0

        lax.fori_loop(0, nchunks // 2, pair_body, 0)
        pltpu.sync_copy(carry, sums_hbm.at[pl.ds(wid * cols, cols)])

    return phase1


def _make_phase2(rows, cols, nw, num_cores, chunk_rows):
    rpw = rows // nw
    nchunks = rpw // chunk_rows
    ngroups = cols // LANES

    mesh = plsc.VectorSubcoreMesh(core_axis_name="c", subcore_axis_name="s")

    @functools.partial(
        pl.kernel,
        out_type=jax.ShapeDtypeStruct((rows, cols), jnp.float32),
        mesh=mesh,
        scratch_types=[
            pltpu.VMEM((chunk_rows, cols), jnp.float32),
            pltpu.VMEM((chunk_rows, cols), jnp.float32),
            pltpu.VMEM((chunk_rows, cols), jnp.float32),
            pltpu.VMEM((chunk_rows, cols), jnp.float32),
            pltpu.VMEM((nw * cols,), jnp.float32),
            pltpu.VMEM((cols,), jnp.float32),
            pltpu.SemaphoreType.DMA,
            pltpu.SemaphoreType.DMA,
            pltpu.SemaphoreType.DMA,
            pltpu.SemaphoreType.DMA,
        ],
        compiler_params=pltpu.CompilerParams(needs_layout_passes=False),
    )
    def phase2(
        x_hbm, sums_hbm, out_hbm, xbuf0, xbuf1, obuf0, obuf1, sums_buf, carry,
        si0, si1, so0, so1,
    ):
        cid = lax.axis_index("c")
        sid = lax.axis_index("s")
        wid = sid * num_cores + cid
        row_base = wid * rpw

        def in_copy(c, buf, sem):
            return pltpu.make_async_copy(
                x_hbm.at[pl.ds(row_base + c * chunk_rows, chunk_rows), :],
                buf,
                sem,
            )

        def out_copy(c, buf, sem):
            return pltpu.make_async_copy(
                buf,
                out_hbm.at[pl.ds(row_base + c * chunk_rows, chunk_rows), :],
                sem,
            )

        in_copy(0, xbuf0, si0).start()

        pltpu.sync_copy(sums_hbm, sums_buf)
        for gg in range(ngroups):
            carry[pl.ds(gg * LANES, LANES)] = jnp.zeros((LANES,), jnp.float32)

        def pref_body(v, _):
            vb = v * cols
            for gg in range(ngroups):
                off = gg * LANES
                carry[pl.ds(off, LANES)] = (
                    carry[pl.ds(off, LANES)] + sums_buf[pl.ds(vb + off, LANES)]
                )
            return 0

        lax.fori_loop(0, wid, pref_body, 0)

        def compute(xbuf, obuf):
            for cg in range(ngroups // CSTRIP):
                col0 = cg * CSTRIP * LANES
                cidx = [_iota16() + (col0 + g * LANES) for g in range(CSTRIP)]
                cs = [
                    carry[pl.ds(col0 + g * LANES, LANES)] for g in range(CSTRIP)
                ]

                def row_body(r, cs, _cidx=cidx):
                    ridx = jnp.full((LANES,), r, jnp.int32)
                    new_cs = []
                    for g in range(CSTRIP):
                        plsc.store_scatter(obuf, [ridx, _cidx[g]], cs[g])
                        new_cs.append(
                            cs[g] + plsc.load_gather(xbuf, [ridx, _cidx[g]])
                        )
                    return tuple(new_cs)

                cs = lax.fori_loop(0, chunk_rows, row_body, tuple(cs), unroll=4)
                for g in range(CSTRIP):
                    carry[pl.ds(col0 + g * LANES, LANES)] = cs[g]

        def pair_body(i, _):
            c0 = 2 * i
            in_copy(c0 + 1, xbuf1, si1).start()
            in_copy(c0, xbuf0, si0).wait()

            @pl.when(i > 0)
            def _():
                out_copy(c0, obuf0, so0).wait()

            compute(xbuf0, obuf0)
            out_copy(c0, obuf0, so0).start()

            @pl.when(c0 + 2 < nchunks)
            def _():
                in_copy(c0 + 2, xbuf0, si0).start()

            in_copy(c0 + 1, xbuf1, si1).wait()

            @pl.when(i > 0)
            def _():
                out_copy(c0 + 1, obuf1, so1).wait()

            compute(xbuf1, obuf1)
            out_copy(c0 + 1, obuf1, so1).start()
            return 0

        lax.fori_loop(0, nchunks // 2, pair_body, 0)
        out_copy(0, obuf0, so0).wait()
        out_copy(1, obuf1, so1).wait()

    return phase2


@jax.jit
def kernel(x):
    rows, cols = x.shape
    info = plsc.get_sparse_core_info()
    nw = info.num_cores * info.num_subcores
    p1 = _make_phase1(rows, cols, nw, info.num_cores, 32)
    p2 = _make_phase2(rows, cols, nw, info.num_cores, 16)
    sums = p1(x)
    return p2(x, sums)


# speedup vs baseline: 1.1372x; 1.1372x over previous
import functools

import jax
import jax.numpy as jnp
from jax import lax
from jax.experimental import pallas as pl
from jax.experimental.pallas import tpu as pltpu
from jax.experimental.pallas import tpu_sc as plsc

LANES = 16
CSTRIP = 16


def _iota16():
    return lax.iota(jnp.int32, LANES)


def _make_phase1(rows, cols, nw, num_cores, chunk_rows):
    rpw = rows // nw
    nchunks = rpw // chunk_rows
    ngroups = cols // LANES

    mesh = plsc.VectorSubcoreMesh(core_axis_name="c", subcore_axis_name="s")

    @functools.partial(
        pl.kernel,
        out_type=jax.ShapeDtypeStruct((nw * cols,), jnp.float32),
        mesh=mesh,
        scratch_types=[
            pltpu.VMEM((chunk_rows, cols), jnp.float32),
            pltpu.VMEM((chunk_rows, cols), jnp.float32),
            pltpu.VMEM((cols,), jnp.float32),
            pltpu.SemaphoreType.DMA,
            pltpu.SemaphoreType.DMA,
        ],
        compiler_params=pltpu.CompilerParams(needs_layout_passes=False),
    )
    def phase1(x_hbm, sums_hbm, xbuf0, xbuf1, carry, s0, s1):
        cid = lax.axis_index("c")
        sid = lax.axis_index("s")
        wid = sid * num_cores + cid
        row_base = wid * rpw

        def in_copy(c, buf, sem):
            return pltpu.make_async_copy(
                x_hbm.at[pl.ds(row_base + c * chunk_rows, chunk_rows), :],
                buf,
                sem,
            )

        def compute(xbuf, first):
            for cg in range(ngroups // CSTRIP):
                col0 = cg * CSTRIP * LANES
                cidx = [_iota16() + (col0 + g * LANES) for g in range(CSTRIP)]
                cs = [
                    jnp.where(
                        first,
                        jnp.zeros((LANES,), jnp.float32),
                        carry[pl.ds(col0 + g * LANES, LANES)],
                    )
                    for g in range(CSTRIP)
                ]

                def row_body(r, cs, _cidx=cidx):
                    ridx = jnp.full((LANES,), r, jnp.int32)
                    return tuple(
                        cs[g] + plsc.load_gather(xbuf, [ridx, _cidx[g]])
                        for g in range(CSTRIP)
                    )

                cs = lax.fori_loop(0, chunk_rows, row_body, tuple(cs), unroll=2)
                for g in range(CSTRIP):
                    carry[pl.ds(col0 + g * LANES, LANES)] = cs[g]

        in_copy(0, xbuf0, s0).start()

        def pair_body(i, _):
            c0 = 2 * i
            in_copy(c0 + 1, xbuf1, s1).start()
            in_copy(c0, xbuf0, s0).wait()
            compute(xbuf0, c0 == 0)

            @pl.when(c0 + 2 < nchunks)
            def _():
                in_copy(c0 + 2, xbuf0, s0).start()

            in_copy(c0 + 1, xbuf1, s1).wait()
            compute(xbuf1, False)
            return 0

        lax.fori_loop(0, nchunks // 2, pair_body, 0)
        pltpu.sync_copy(carry, sums_hbm.at[pl.ds(wid * cols, cols)])

    return phase1


def _make_phase2(rows, cols, nw, num_cores, chunk_rows):
    rpw = rows // nw
    nchunks = rpw // chunk_rows
    ngroups = cols // LANES

    mesh = plsc.VectorSubcoreMesh(core_axis_name="c", subcore_axis_name="s")

    @functools.partial(
        pl.kernel,
        out_type=jax.ShapeDtypeStruct((rows, cols), jnp.float32),
        mesh=mesh,
        scratch_types=[
            pltpu.VMEM((chunk_rows, cols), jnp.float32),
            pltpu.VMEM((chunk_rows, cols), jnp.float32),
            pltpu.VMEM((chunk_rows, cols), jnp.float32),
            pltpu.VMEM((chunk_rows, cols), jnp.float32),
            pltpu.VMEM((nw * cols,), jnp.float32),
            pltpu.VMEM((cols,), jnp.float32),
            pltpu.SemaphoreType.DMA,
            pltpu.SemaphoreType.DMA,
            pltpu.SemaphoreType.DMA,
            pltpu.SemaphoreType.DMA,
        ],
        compiler_params=pltpu.CompilerParams(needs_layout_passes=False),
    )
    def phase2(
        x_hbm, sums_hbm, out_hbm, xbuf0, xbuf1, obuf0, obuf1, sums_buf, carry,
        si0, si1, so0, so1,
    ):
        cid = lax.axis_index("c")
        sid = lax.axis_index("s")
        wid = sid * num_cores + cid
        row_base = wid * rpw

        def in_copy(c, buf, sem):
            return pltpu.make_async_copy(
                x_hbm.at[pl.ds(row_base + c * chunk_rows, chunk_rows), :],
                buf,
                sem,
            )

        def out_copy(c, buf, sem):
            return pltpu.make_async_copy(
                buf,
                out_hbm.at[pl.ds(row_base + c * chunk_rows, chunk_rows), :],
                sem,
            )

        in_copy(0, xbuf0, si0).start()

        pltpu.sync_copy(sums_hbm, sums_buf)
        for gg in range(ngroups):
            carry[pl.ds(gg * LANES, LANES)] = jnp.zeros((LANES,), jnp.float32)

        def pref_body(v, _):
            vb = v * cols
            for gg in range(ngroups):
                off = gg * LANES
                carry[pl.ds(off, LANES)] = (
                    carry[pl.ds(off, LANES)] + sums_buf[pl.ds(vb + off, LANES)]
                )
            return 0

        lax.fori_loop(0, wid, pref_body, 0)

        def compute(xbuf, obuf):
            for cg in range(ngroups // CSTRIP):
                col0 = cg * CSTRIP * LANES
                cidx = [_iota16() + (col0 + g * LANES) for g in range(CSTRIP)]
                cs = [
                    carry[pl.ds(col0 + g * LANES, LANES)] for g in range(CSTRIP)
                ]

                def row_body(r, cs, _cidx=cidx):
                    ridx = jnp.full((LANES,), r, jnp.int32)
                    new_cs = []
                    for g in range(CSTRIP):
                        plsc.store_scatter(obuf, [ridx, _cidx[g]], cs[g])
                        new_cs.append(
                            cs[g] + plsc.load_gather(xbuf, [ridx, _cidx[g]])
                        )
                    return tuple(new_cs)

                cs = lax.fori_loop(0, chunk_rows, row_body, tuple(cs), unroll=2)
                for g in range(CSTRIP):
                    carry[pl.ds(col0 + g * LANES, LANES)] = cs[g]

        def pair_body(i, _):
            c0 = 2 * i
            in_copy(c0 + 1, xbuf1, si1).start()
            in_copy(c0, xbuf0, si0).wait()

            @pl.when(i > 0)
            def _():
                out_copy(c0, obuf0, so0).wait()

            compute(xbuf0, obuf0)
            out_copy(c0, obuf0, so0).start()

            @pl.when(c0 + 2 < nchunks)
            def _():
                in_copy(c0 + 2, xbuf0, si0).start()

            in_copy(c0 + 1, xbuf1, si1).wait()

            @pl.when(i > 0)
            def _():
                out_copy(c0 + 1, obuf1, so1).wait()

            compute(xbuf1, obuf1)
            out_copy(c0 + 1, obuf1, so1).start()
            return 0

        lax.fori_loop(0, nchunks // 2, pair_body, 0)
        out_copy(0, obuf0, so0).wait()
        out_copy(1, obuf1, so1).wait()

    return phase2


@jax.jit
def kernel(x):
    rows, cols = x.shape
    info = plsc.get_sparse_core_info()
    nw = info.num_cores * info.num_subcores
    p1 = _make_phase1(rows, cols, nw, info.num_cores, 32)
    p2 = _make_phase2(rows, cols, nw, info.num_cores, 16)
    sums = p1(x)
    return p2(x, sums)


# speedup vs baseline: 1.1703x; 1.0292x over previous
import functools

import jax
import jax.numpy as jnp
from jax import lax
from jax.experimental import pallas as pl
from jax.experimental.pallas import tpu as pltpu
from jax.experimental.pallas import tpu_sc as plsc

LANES = 16
CSTRIP = 16


def _iota16():
    return lax.iota(jnp.int32, LANES)


def _make_scan(rows, cols, num_cores, num_subcores, chunk_rows):
    chalf = cols // num_cores
    rpw = rows // num_subcores
    nchunks = rpw // chunk_rows
    ngroups = chalf // LANES

    mesh = plsc.VectorSubcoreMesh(core_axis_name="c", subcore_axis_name="s")

    @functools.partial(
        pl.kernel,
        out_type=jax.ShapeDtypeStruct((rows, cols), jnp.float32),
        mesh=mesh,
        scratch_types=[
            pltpu.VMEM((chunk_rows, chalf), jnp.float32),
            pltpu.VMEM((chunk_rows, chalf), jnp.float32),
            pltpu.VMEM((chunk_rows, chalf), jnp.float32),
            pltpu.VMEM((chunk_rows, chalf), jnp.float32),
            pltpu.VMEM((num_subcores * chalf,), jnp.float32),
            pltpu.VMEM((chalf,), jnp.float32),
            pltpu.VMEM_SHARED((num_subcores * chalf,), jnp.float32),
            pltpu.SemaphoreType.DMA,
            pltpu.SemaphoreType.DMA,
            pltpu.SemaphoreType.DMA,
            pltpu.SemaphoreType.DMA,
        ],
        compiler_params=pltpu.CompilerParams(needs_layout_passes=False),
    )
    def scan(
        x_hbm, out_hbm, xbuf0, xbuf1, obuf0, obuf1, sums_buf, carry, shared,
        si0, si1, so0, so1,
    ):
        cid = lax.axis_index("c")
        sid = lax.axis_index("s")
        row_base = sid * rpw
        col_base = cid * chalf

        def in_copy(c, buf, sem):
            return pltpu.make_async_copy(
                x_hbm.at[
                    pl.ds(row_base + c * chunk_rows, chunk_rows),
                    pl.ds(col_base, chalf),
                ],
                buf,
                sem,
            )

        def out_copy(c, buf, sem):
            return pltpu.make_async_copy(
                buf,
                out_hbm.at[
                    pl.ds(row_base + c * chunk_rows, chunk_rows),
                    pl.ds(col_base, chalf),
                ],
                sem,
            )

        def compute_a(xbuf, first):
            for cg in range(ngroups // CSTRIP):
                col0 = cg * CSTRIP * LANES
                cidx = [_iota16() + (col0 + g * LANES) for g in range(CSTRIP)]
                cs = [
                    jnp.where(
                        first,
                        jnp.zeros((LANES,), jnp.float32),
                        carry[pl.ds(col0 + g * LANES, LANES)],
                    )
                    for g in range(CSTRIP)
                ]

                def row_body(r, cs, _cidx=cidx):
                    ridx = jnp.full((LANES,), r, jnp.int32)
                    return tuple(
                        cs[g] + plsc.load_gather(xbuf, [ridx, _cidx[g]])
                        for g in range(CSTRIP)
                    )

                cs = lax.fori_loop(0, chunk_rows, row_body, tuple(cs), unroll=2)
                for g in range(CSTRIP):
                    carry[pl.ds(col0 + g * LANES, LANES)] = cs[g]

        def compute_b(xbuf, obuf):
            for cg in range(ngroups // CSTRIP):
                col0 = cg * CSTRIP * LANES
                cidx = [_iota16() + (col0 + g * LANES) for g in range(CSTRIP)]
                cs = [
                    carry[pl.ds(col0 + g * LANES, LANES)] for g in range(CSTRIP)
                ]

                def row_body(r, cs, _cidx=cidx):
                    ridx = jnp.full((LANES,), r, jnp.int32)
                    new_cs = []
                    for g in range(CSTRIP):
                        plsc.store_scatter(obuf, [ridx, _cidx[g]], cs[g])
                        new_cs.append(
                            cs[g] + plsc.load_gather(xbuf, [ridx, _cidx[g]])
                        )
                    return tuple(new_cs)

                cs = lax.fori_loop(0, chunk_rows, row_body, tuple(cs), unroll=2)
                for g in range(CSTRIP):
                    carry[pl.ds(col0 + g * LANES, LANES)] = cs[g]

        in_copy(0, xbuf0, si0).start()
        in_copy(1, xbuf1, si1).start()

        def pair_a(i, _):
            c0 = 2 * i
            in_copy(c0, xbuf0, si0).wait()
            compute_a(xbuf0, c0 == 0)

            @pl.when(c0 + 2 < nchunks)
            def _():
                in_copy(c0 + 2, xbuf0, si0).start()

            in_copy(c0 + 1, xbuf1, si1).wait()
            compute_a(xbuf1, False)

            @pl.when(c0 + 3 < nchunks)
            def _():
                in_copy(c0 + 3, xbuf1, si1).start()

            return 0

        lax.fori_loop(0, nchunks // 2, pair_a, 0)

        in_copy(0, xbuf0, si0).start()
        in_copy(1, xbuf1, si1).start()

        pltpu.sync_copy(carry, shared.at[pl.ds(sid * chalf, chalf)])
        plsc.subcore_barrier()
        pltpu.sync_copy(shared, sums_buf)

        for gg in range(ngroups):
            carry[pl.ds(gg * LANES, LANES)] = jnp.zeros((LANES,), jnp.float32)

        def pref_body(v, _):
            vb = v * chalf
            for gg in range(ngroups):
                off = gg * LANES
                carry[pl.ds(off, LANES)] = (
                    carry[pl.ds(off, LANES)] + sums_buf[pl.ds(vb + off, LANES)]
                )
            return 0

        lax.fori_loop(0, sid, pref_body, 0)

        def pair_b(i, _):
            c0 = 2 * i
            in_copy(c0, xbuf0, si0).wait()

            @pl.when(i > 0)
            def _():
                out_copy(c0, obuf0, so0).wait()

            compute_b(xbuf0, obuf0)
            out_copy(c0, obuf0, so0).start()

            @pl.when(c0 + 2 < nchunks)
            def _():
                in_copy(c0 + 2, xbuf0, si0).start()

            in_copy(c0 + 1, xbuf1, si1).wait()

            @pl.when(i > 0)
            def _():
                out_copy(c0 + 1, obuf1, so1).wait()

            compute_b(xbuf1, obuf1)
            out_copy(c0 + 1, obuf1, so1).start()

            @pl.when(c0 + 3 < nchunks)
            def _():
                in_copy(c0 + 3, xbuf1, si1).start()

            return 0

        lax.fori_loop(0, nchunks // 2, pair_b, 0)
        out_copy(0, obuf0, so0).wait()
        out_copy(1, obuf1, so1).wait()

    return scan


@jax.jit
def kernel(x):
    rows, cols = x.shape
    info = plsc.get_sparse_core_info()
    fn = _make_scan(rows, cols, info.num_cores, info.num_subcores, 32)
    return fn(x)
